# baseline (speedup 1.0000x reference)
import functools

import jax
import jax.numpy as jnp
from jax import lax
from jax.experimental import pallas as pl
from jax.experimental.pallas import tpu as pltpu
from jax.experimental.pallas import tpu_sc as plsc

NUM_EMBEDDINGS = 1000000
EMBED_DIM = 3
B_ROWS = 16384
B_COLS = 26
TOTAL = B_ROWS * B_COLS
NUM_WORKERS = 32
PER_W = TOTAL // NUM_WORKERS
CHUNK = 128
NCHUNKS = PER_W // CHUNK


def _gather_body(tab_hbm, idx_hbm, out_hbm, idx_v, rows_v, sem):
    wid = lax.axis_index("s") * 2 + lax.axis_index("c")
    base = wid * PER_W
    pltpu.sync_copy(idx_hbm.at[pl.ds(base, PER_W)], idx_v)

    def chunk(j, carry):
        off = j * CHUNK
        cp = pltpu.async_copy(
            tab_hbm.at[idx_v.at[pl.ds(off, CHUNK)]],
            rows_v.at[pl.ds(off, CHUNK)],
            sem,
        )
        cp.wait()
        return carry

    lax.fori_loop(0, NCHUNKS, chunk, 0)
    pltpu.sync_copy(rows_v, out_hbm.at[pl.ds(base, PER_W)])


_gather_call = pl.kernel(
    _gather_body,
    out_type=jax.ShapeDtypeStruct((TOTAL, EMBED_DIM), jnp.float32),
    mesh=plsc.VectorSubcoreMesh(core_axis_name="c", subcore_axis_name="s"),
    scratch_types=[
        pltpu.VMEM((PER_W,), jnp.int32),
        pltpu.VMEM((PER_W, EMBED_DIM), jnp.float32),
        pltpu.SemaphoreType.DMA,
    ],
    compiler_params=pltpu.CompilerParams(use_tc_tiling_on_sc=False),
)


@jax.jit
def kernel(inputs, weight):
    flat_idx = inputs.reshape(TOTAL).astype(jnp.int32)
    out = _gather_call(weight, flat_idx)
    return out.reshape(B_ROWS, B_COLS, EMBED_DIM)


# speedup vs baseline: 1.0162x; 1.0162x over previous
import functools

import jax
import jax.numpy as jnp
from jax import lax
from jax.experimental import pallas as pl
from jax.experimental.pallas import tpu as pltpu
from jax.experimental.pallas import tpu_sc as plsc

NUM_EMBEDDINGS = 1000000
EMBED_DIM = 3
B_ROWS = 16384
B_COLS = 26
TOTAL = B_ROWS * B_COLS
NUM_WORKERS = 32
PER_W = TOTAL // NUM_WORKERS
CHUNK = 128
NCHUNKS = PER_W // CHUNK


def _gather_body(tab_hbm, idx_hbm, out_hbm, idx_v, rows_v, sem):
    wid = lax.axis_index("s") * 2 + lax.axis_index("c")
    base = wid * PER_W
    pltpu.sync_copy(idx_hbm.at[pl.ds(base, PER_W)], idx_v)
    pltpu.async_copy(tab_hbm.at[idx_v], rows_v, sem).wait()
    pltpu.sync_copy(rows_v, out_hbm.at[pl.ds(base, PER_W)])


_gather_call = pl.kernel(
    _gather_body,
    out_type=jax.ShapeDtypeStruct((TOTAL, EMBED_DIM), jnp.float32),
    mesh=plsc.VectorSubcoreMesh(core_axis_name="c", subcore_axis_name="s"),
    scratch_types=[
        pltpu.VMEM((PER_W,), jnp.int32),
        pltpu.VMEM((PER_W, EMBED_DIM), jnp.float32),
        pltpu.SemaphoreType.DMA,
    ],
    compiler_params=pltpu.CompilerParams(use_tc_tiling_on_sc=False),
)


@jax.jit
def kernel(inputs, weight):
    flat_idx = inputs.reshape(TOTAL).astype(jnp.int32)
    out = _gather_call(weight, flat_idx)
    return out.reshape(B_ROWS, B_COLS, EMBED_DIM)


# speedup vs baseline: 1.0795x; 1.0623x over previous
import functools

import jax
import jax.numpy as jnp
from jax import lax
from jax.experimental import pallas as pl
from jax.experimental.pallas import tpu as pltpu
from jax.experimental.pallas import tpu_sc as plsc

NUM_EMBEDDINGS = 1000000
EMBED_DIM = 3
B_ROWS = 16384
B_COLS = 26
TOTAL = B_ROWS * B_COLS
NUM_WORKERS = 32
PER_W = TOTAL // NUM_WORKERS
WORDS_W = PER_W * EMBED_DIM
TABLE_WORDS = NUM_EMBEDDINGS * EMBED_DIM
LANES = 16


def _gather_body(tab_hbm, idx_hbm, out_hbm, idx_v, widx_v, rows_v, sem):
    wid = lax.axis_index("s") * 2 + lax.axis_index("c")
    base = wid * PER_W
    pltpu.sync_copy(idx_hbm.at[pl.ds(base, PER_W)], idx_v)

    iota = lax.iota(jnp.int32, LANES)
    iota3 = iota * 3

    def expand(g, carry):
        k0 = g * LANES
        iv = idx_v[pl.ds(k0, LANES)]
        t0 = iv * 3
        p0 = jnp.full((LANES,), k0 * 3, dtype=jnp.int32) + iota3
        plsc.store_scatter(widx_v, [p0], t0)
        plsc.store_scatter(widx_v, [p0 + 1], t0 + 1)
        plsc.store_scatter(widx_v, [p0 + 2], t0 + 2)
        return carry

    lax.fori_loop(0, PER_W // LANES, expand, 0)

    pltpu.async_copy(tab_hbm.at[widx_v], rows_v, sem).wait()
    pltpu.sync_copy(rows_v, out_hbm.at[pl.ds(base * EMBED_DIM, WORDS_W)])


_gather_call = pl.kernel(
    _gather_body,
    out_type=jax.ShapeDtypeStruct((TOTAL * EMBED_DIM,), jnp.float32),
    mesh=plsc.VectorSubcoreMesh(core_axis_name="c", subcore_axis_name="s"),
    scratch_types=[
        pltpu.VMEM((PER_W,), jnp.int32),
        pltpu.VMEM((WORDS_W,), jnp.int32),
        pltpu.VMEM((WORDS_W,), jnp.float32),
        pltpu.SemaphoreType.DMA,
    ],
    compiler_params=pltpu.CompilerParams(
        use_tc_tiling_on_sc=False, needs_layout_passes=False
    ),
)


@jax.jit
def kernel(inputs, weight):
    flat_idx = inputs.reshape(TOTAL).astype(jnp.int32)
    flat_w = weight.reshape(TABLE_WORDS)
    out = _gather_call(flat_w, flat_idx)
    return out.reshape(B_ROWS, B_COLS, EMBED_DIM)


# speedup vs baseline: 8.7314x; 8.0884x over previous
import functools

import jax
import jax.numpy as jnp
from jax import lax
from jax.experimental import pallas as pl
from jax.experimental.pallas import tpu as pltpu
from jax.experimental.pallas import tpu_sc as plsc

NUM_EMBEDDINGS = 1000000
EMBED_DIM = 3
B_ROWS = 16384
B_COLS = 26
TOTAL = B_ROWS * B_COLS
NUM_WORKERS = 32
PER_W = TOTAL // NUM_WORKERS
WORDS_W = PER_W * EMBED_DIM
TABLE_WORDS = NUM_EMBEDDINGS * EMBED_DIM
LANES = 16


def _gather_body(tab_hbm, idx_hbm, out_hbm, idx_v, widx_v, rows_v, sem):
    wid = lax.axis_index("s") * 2 + lax.axis_index("c")
    base = wid * PER_W
    pltpu.sync_copy(idx_hbm.at[pl.ds(base, PER_W)], idx_v)

    iota = lax.iota(jnp.int32, LANES)
    iota3 = iota * 3

    def expand(g, carry):
        k0 = g * LANES
        iv = idx_v[pl.ds(k0, LANES)]
        p0 = jnp.full((LANES,), k0 * 3, dtype=jnp.int32) + iota3
        plsc.store_scatter(widx_v, [p0], iv)
        plsc.store_scatter(widx_v, [p0 + 1], iv + NUM_EMBEDDINGS)
        plsc.store_scatter(widx_v, [p0 + 2], iv + 2 * NUM_EMBEDDINGS)
        return carry

    lax.fori_loop(0, PER_W // LANES, expand, 0)

    pltpu.async_copy(tab_hbm.at[widx_v], rows_v, sem).wait()
    pltpu.sync_copy(rows_v, out_hbm.at[pl.ds(base * EMBED_DIM, WORDS_W)])


_gather_call = pl.kernel(
    _gather_body,
    out_type=jax.ShapeDtypeStruct((TOTAL * EMBED_DIM,), jnp.float32),
    mesh=plsc.VectorSubcoreMesh(core_axis_name="c", subcore_axis_name="s"),
    scratch_types=[
        pltpu.VMEM((PER_W,), jnp.int32),
        pltpu.VMEM((WORDS_W,), jnp.int32),
        pltpu.VMEM((WORDS_W,), jnp.float32),
        pltpu.SemaphoreType.DMA,
    ],
    compiler_params=pltpu.CompilerParams(
        use_tc_tiling_on_sc=False, needs_layout_passes=False
    ),
)


@jax.jit
def kernel(inputs, weight):
    flat_idx = inputs.reshape(TOTAL).astype(jnp.int32)
    flat_w = weight.T.reshape(TABLE_WORDS)
    out = _gather_call(flat_w, flat_idx)
    return out.reshape(B_ROWS, B_COLS, EMBED_DIM)


# speedup vs baseline: 32.0048x; 3.6655x over previous
import functools

import jax
import jax.numpy as jnp
from jax import lax
from jax.experimental import pallas as pl
from jax.experimental.pallas import tpu as pltpu
from jax.experimental.pallas import tpu_sc as plsc

NUM_EMBEDDINGS = 1000000
EMBED_DIM = 3
B_ROWS = 16384
B_COLS = 26
TOTAL = B_ROWS * B_COLS
TABLE_WORDS = NUM_EMBEDDINGS * EMBED_DIM
NUM_WORKERS = 32
BPW = B_ROWS // NUM_WORKERS
ROWS = EMBED_DIM * B_COLS
LANES = 16
GRP = BPW // LANES


def _gather_body(tab_hbm, idx_hbm, out_hbm, idx_v, widx_v, rows_v, sem, osem):
    wid = lax.axis_index("s") * 2 + lax.axis_index("c")
    b0 = wid * BPW

    pltpu.sync_copy(idx_hbm.at[:, pl.ds(b0, BPW)], idx_v)

    def expand(r, carry):
        j = r % B_COLS
        plane = (r // B_COLS) * NUM_EMBEDDINGS

        def grp(g, c2):
            widx_v[r, pl.ds(g * LANES, LANES)] = (
                idx_v[j, pl.ds(g * LANES, LANES)] + plane
            )
            return c2

        lax.fori_loop(0, GRP, grp, 0)
        return carry

    lax.fori_loop(0, ROWS, expand, 0)

    def fire(r, carry):
        pltpu.async_copy(tab_hbm.at[widx_v.at[r]], rows_v.at[r], sem)
        return carry

    lax.fori_loop(0, ROWS, fire, 0)

    def drain(r, carry):
        pltpu.make_async_copy(tab_hbm.at[widx_v.at[r]], rows_v.at[r], sem).wait()
        return carry

    lax.fori_loop(0, ROWS, drain, 0)

    pltpu.sync_copy(rows_v, out_hbm.at[:, pl.ds(b0, BPW)])


_gather_call = pl.kernel(
    _gather_body,
    out_type=jax.ShapeDtypeStruct((ROWS, B_ROWS), jnp.float32),
    mesh=plsc.VectorSubcoreMesh(core_axis_name="c", subcore_axis_name="s"),
    scratch_types=[
        pltpu.VMEM((B_COLS, BPW), jnp.int32),
        pltpu.VMEM((ROWS, BPW), jnp.int32),
        pltpu.VMEM((ROWS, BPW), jnp.float32),
        pltpu.SemaphoreType.DMA,
        pltpu.SemaphoreType.DMA,
    ],
    compiler_params=pltpu.CompilerParams(
        use_tc_tiling_on_sc=False, needs_layout_passes=False
    ),
)


@jax.jit
def kernel(inputs, weight):
    flat_w = weight.T.reshape(TABLE_WORDS)
    idx_cm = inputs.T.astype(jnp.int32)
    out = _gather_call(flat_w, idx_cm)
    return out.reshape(EMBED_DIM, B_COLS, B_ROWS).transpose(2, 1, 0)
